# baseline (speedup 1.0000x reference)
import functools
import jax
import jax.numpy as jnp
from jax import lax
from jax.experimental import pallas as pl
from jax.experimental.pallas import tpu as pltpu
from jax.experimental.pallas import tpu_sc as plsc

NH = 12
NPAD = 32
LL = 16


def _scores_body(q_ref, w_ref, out_ref):
    parts = []
    for h in range(NH):
        parts.append(jnp.dot(q_ref[h], w_ref[h],
                             preferred_element_type=jnp.float32))
    out_ref[...] = jnp.concatenate(parts, axis=-1)


def _compute_scores(qv, wt):
    Q = qv.shape[1]
    return pl.pallas_call(
        _scores_body,
        out_shape=jax.ShapeDtypeStruct((Q, NH * NPAD), jnp.float32),
    )(qv, wt)


def _make_sc_gather(Q, K):
    info = plsc.get_sparse_core_info()
    NC, NS = info.num_cores, info.num_subcores
    NW = NC * NS
    QW = Q // NW
    mesh = plsc.VectorSubcoreMesh(core_axis_name="c", subcore_axis_name="s")

    @functools.partial(
        pl.kernel,
        mesh=mesh,
        compiler_params=pltpu.CompilerParams(needs_layout_passes=False),
        out_type=jax.ShapeDtypeStruct((NH, Q, K), jnp.float32),
        scratch_types=[
            pltpu.VMEM((QW, NH * NPAD), jnp.float32),
            pltpu.VMEM((2, K), jnp.int32),
            pltpu.VMEM((2, NH, K), jnp.float32),
            pltpu.SemaphoreType.DMA,
            pltpu.SemaphoreType.DMA,
            pltpu.SemaphoreType.DMA,
            pltpu.SemaphoreType.DMA,
        ],
    )
    def sc_gather(scores_hbm, rp_hbm, out_hbm, tbl_v, idx_v, obuf_v,
                  si0, si1, so0, so1):
        sin = (si0, si1)
        sout = (so0, so1)
        wid = lax.axis_index("s") * NC + lax.axis_index("c")
        base = wid * QW
        pltpu.sync_copy(scores_hbm.at[pl.ds(base, QW)], tbl_v)

        def idx_dma(qi, b):
            return pltpu.make_async_copy(
                rp_hbm.at[base + qi], idx_v.at[b], sin[b])

        def out_dma(qi, b):
            return pltpu.make_async_copy(
                obuf_v.at[b], out_hbm.at[:, base + qi], sout[b])

        idx_dma(0, 0).start()
        idx_dma(1, 1).start()

        def q_pair(p, carry):
            for b in range(2):
                qi = p * 2 + b
                idx_dma(qi, b).wait()

                @pl.when(qi >= 2)
                def _():
                    out_dma(qi - 2, b).wait()

                @plsc.parallel_loop(0, K // LL, unroll=2)
                def _(c):
                    iv = idx_v[b, pl.ds(c * LL, LL)]
                    for h in range(NH):
                        obuf_v[b, h, pl.ds(c * LL, LL)] = plsc.load_gather(
                            tbl_v.at[qi, pl.ds(h * NPAD, NPAD)], [iv])
                out_dma(qi, b).start()

                @pl.when(qi + 2 < QW)
                def _():
                    idx_dma(qi + 2, b).start()
            return carry

        lax.fori_loop(0, QW // 2, q_pair, 0)
        out_dma(QW - 2, 0).wait()
        out_dma(QW - 1, 1).wait()

    return sc_gather


def kernel(query, relpos, emb_weight):
    B, H, Q, dh = query.shape
    K = relpos.shape[2]
    n_emb = emb_weight.shape[0]

    wt = emb_weight.at[0].set(0.0).reshape(n_emb, H, dh).transpose(1, 2, 0)
    wt = jnp.pad(wt, ((0, 0), (0, 0), (0, NPAD - n_emb)))
    qv = query.reshape(H, Q, dh)

    scores = _compute_scores(qv, wt)
    rp = relpos.reshape(Q, K)
    out = _make_sc_gather(Q, K)(scores, rp)
    return out.reshape(B, H, Q, K)


# speedup vs baseline: 1.1004x; 1.1004x over previous
import functools
import jax
import jax.numpy as jnp
from jax import lax
from jax.experimental import pallas as pl
from jax.experimental.pallas import tpu as pltpu
from jax.experimental.pallas import tpu_sc as plsc

NH = 12
NPAD = 32
LL = 16


def _scores_body(q_ref, w_ref, out_ref):
    parts = []
    for h in range(NH):
        parts.append(jnp.dot(q_ref[h], w_ref[h],
                             preferred_element_type=jnp.float32))
    out_ref[...] = jnp.concatenate(parts, axis=-1)


def _compute_scores(qv, wt):
    Q = qv.shape[1]
    return pl.pallas_call(
        _scores_body,
        out_shape=jax.ShapeDtypeStruct((Q, NH * NPAD), jnp.float32),
    )(qv, wt)


def _make_sc_gather(Q, K):
    info = plsc.get_sparse_core_info()
    NC, NS = info.num_cores, info.num_subcores
    NW = NC * NS
    QW = Q // NW
    mesh = plsc.VectorSubcoreMesh(core_axis_name="c", subcore_axis_name="s")

    @functools.partial(
        pl.kernel,
        mesh=mesh,
        compiler_params=pltpu.CompilerParams(needs_layout_passes=False),
        out_type=jax.ShapeDtypeStruct((NH, Q, K), jnp.float32),
        scratch_types=[
            pltpu.VMEM((QW, NH * NPAD), jnp.float32),
            pltpu.VMEM((2, K), jnp.int32),
            pltpu.VMEM((2, NH, K), jnp.float32),
            pltpu.SemaphoreType.DMA,
            pltpu.SemaphoreType.DMA,
            pltpu.SemaphoreType.DMA,
            pltpu.SemaphoreType.DMA,
        ],
    )
    def sc_gather(scores_hbm, rp_hbm, out_hbm, tbl_v, idx_v, obuf_v,
                  si0, si1, so0, so1):
        sin = (si0, si1)
        sout = (so0, so1)
        wid = lax.axis_index("s") * NC + lax.axis_index("c")
        base = wid * QW
        pltpu.sync_copy(scores_hbm.at[pl.ds(base, QW)], tbl_v)

        def idx_dma(qi, b):
            return pltpu.make_async_copy(
                rp_hbm.at[pl.ds((base + qi) * K, K)], idx_v.at[b], sin[b])

        def out_dma(qi, b):
            return pltpu.make_async_copy(
                obuf_v.at[b], out_hbm.at[:, base + qi], sout[b])

        idx_dma(0, 0).start()
        idx_dma(1, 1).start()

        def q_pair(p, carry):
            for b in range(2):
                qi = p * 2 + b
                idx_dma(qi, b).wait()

                @pl.when(qi >= 2)
                def _():
                    out_dma(qi - 2, b).wait()

                @plsc.parallel_loop(0, K // LL, unroll=2)
                def _(c):
                    iv = idx_v[b, pl.ds(c * LL, LL)]
                    for h in range(NH):
                        obuf_v[b, h, pl.ds(c * LL, LL)] = plsc.load_gather(
                            tbl_v.at[qi, pl.ds(h * NPAD, NPAD)], [iv])
                out_dma(qi, b).start()

                @pl.when(qi + 2 < QW)
                def _():
                    idx_dma(qi + 2, b).start()
            return carry

        lax.fori_loop(0, QW // 2, q_pair, 0)
        out_dma(QW - 2, 0).wait()
        out_dma(QW - 1, 1).wait()

    return sc_gather


def kernel(query, relpos, emb_weight):
    B, H, Q, dh = query.shape
    K = relpos.shape[2]
    n_emb = emb_weight.shape[0]

    wt = emb_weight.at[0].set(0.0).reshape(n_emb, H, dh).transpose(1, 2, 0)
    wt = jnp.pad(wt, ((0, 0), (0, 0), (0, NPAD - n_emb)))
    qv = query.reshape(H, Q, dh)

    scores = _compute_scores(qv, wt)
    rp = relpos.reshape(Q * K)
    out = _make_sc_gather(Q, K)(scores, rp)
    return out.reshape(B, H, Q, K)


# speedup vs baseline: 1.1491x; 1.0443x over previous
import functools
import jax
import jax.numpy as jnp
from jax import lax
from jax.experimental import pallas as pl
from jax.experimental.pallas import tpu as pltpu
from jax.experimental.pallas import tpu_sc as plsc

NH = 12
NPAD = 64
LL = 16


def _scores_body(q_ref, w_ref, out_ref):
    parts = []
    for p in range(NH // 2):
        a = jnp.dot(q_ref[2 * p], w_ref[2 * p],
                    preferred_element_type=jnp.float32)
        b = jnp.dot(q_ref[2 * p + 1], w_ref[2 * p + 1],
                    preferred_element_type=jnp.float32)
        ah = lax.bitcast_convert_type(
            a.astype(jnp.bfloat16).astype(jnp.float32), jnp.uint32)
        bh = lax.bitcast_convert_type(
            b.astype(jnp.bfloat16).astype(jnp.float32), jnp.uint32)
        parts.append(
            lax.bitcast_convert_type(ah | (bh >> 16), jnp.int32))
    out_ref[...] = jnp.concatenate(parts, axis=-1)


def _compute_scores(qv, wt):
    Q = qv.shape[1]
    return pl.pallas_call(
        _scores_body,
        out_shape=jax.ShapeDtypeStruct((Q, NH // 2 * NPAD), jnp.int32),
    )(qv, wt)


def _make_sc_gather(Q, K):
    info = plsc.get_sparse_core_info()
    NC, NS = info.num_cores, info.num_subcores
    NW = NC * NS
    QW = Q // NW
    mesh = plsc.VectorSubcoreMesh(core_axis_name="c", subcore_axis_name="s")

    @functools.partial(
        pl.kernel,
        mesh=mesh,
        compiler_params=pltpu.CompilerParams(needs_layout_passes=False),
        out_type=jax.ShapeDtypeStruct((NH, Q, K), jnp.float32),
        scratch_types=[
            pltpu.VMEM((QW, NH // 2 * NPAD), jnp.int32),
            pltpu.VMEM((2, K), jnp.int32),
            pltpu.VMEM((2, NH, K), jnp.float32),
            pltpu.SemaphoreType.DMA,
            pltpu.SemaphoreType.DMA,
            pltpu.SemaphoreType.DMA,
            pltpu.SemaphoreType.DMA,
        ],
    )
    def sc_gather(scores_hbm, rp_hbm, out_hbm, tbl_v, idx_v, obuf_v,
                  si0, si1, so0, so1):
        sin = (si0, si1)
        sout = (so0, so1)
        wid = lax.axis_index("s") * NC + lax.axis_index("c")
        base = wid * QW
        pltpu.sync_copy(scores_hbm.at[pl.ds(base, QW)], tbl_v)

        def idx_dma(qi, b):
            return pltpu.make_async_copy(
                rp_hbm.at[pl.ds((base + qi) * K, K)], idx_v.at[b], sin[b])

        def out_dma(qi, b):
            return pltpu.make_async_copy(
                obuf_v.at[b], out_hbm.at[:, base + qi], sout[b])

        idx_dma(0, 0).start()
        idx_dma(1, 1).start()

        def q_pair(p, carry):
            for b in range(2):
                qi = p * 2 + b
                idx_dma(qi, b).wait()

                @pl.when(qi >= 2)
                def _():
                    out_dma(qi - 2, b).wait()

                @plsc.parallel_loop(0, K // LL, unroll=2)
                def _(c):
                    iv = idx_v[b, pl.ds(c * LL, LL)]
                    for p in range(NH // 2):
                        g = plsc.load_gather(
                            tbl_v.at[qi, pl.ds(p * NPAD, NPAD)], [iv])
                        hi = jnp.bitwise_and(g, jnp.int32(-65536))
                        lo = lax.shift_left(g, jnp.int32(16))
                        obuf_v[b, 2 * p, pl.ds(c * LL, LL)] = plsc.bitcast(
                            hi, jnp.float32)
                        obuf_v[b, 2 * p + 1, pl.ds(c * LL, LL)] = plsc.bitcast(
                            lo, jnp.float32)
                out_dma(qi, b).start()

                @pl.when(qi + 2 < QW)
                def _():
                    idx_dma(qi + 2, b).start()
            return carry

        lax.fori_loop(0, QW // 2, q_pair, 0)
        out_dma(QW - 2, 0).wait()
        out_dma(QW - 1, 1).wait()

    return sc_gather


def kernel(query, relpos, emb_weight):
    B, H, Q, dh = query.shape
    K = relpos.shape[2]
    n_emb = emb_weight.shape[0]

    wt = emb_weight.at[0].set(0.0).reshape(n_emb, H, dh).transpose(1, 2, 0)
    wt = jnp.pad(wt, ((0, 0), (0, 0), (0, NPAD - n_emb)))
    qv = query.reshape(H, Q, dh)

    scores = _compute_scores(qv, wt)
    rp = relpos.reshape(Q * K)
    out = _make_sc_gather(Q, K)(scores, rp)
    return out.reshape(B, H, Q, K)
